# baseline (speedup 1.0000x reference)
import functools

import jax
import jax.numpy as jnp
from jax import lax
from jax.experimental import pallas as pl
from jax.experimental.pallas import tpu as pltpu
from jax.experimental.pallas import tpu_sc as plsc

_B = 16384
_F = 26
_V = 100001
_D = 16

_NC = 2
_NS = 16
_NW = _NC * _NS
_N = _B * _F
_PER_W = _N // _NW
_GSZ = 128
_CHUNK = 512
_NG = _CHUNK // _GSZ
_NCHUNK = _PER_W // _CHUNK


def _sc_gather(idx_hbm, tab_hbm):
    mesh = plsc.VectorSubcoreMesh(core_axis_name="c", subcore_axis_name="s")

    @functools.partial(
        pl.kernel,
        mesh=mesh,
        out_type=jax.ShapeDtypeStruct((_N, _D), jnp.float32),
        compiler_params=pltpu.CompilerParams(use_tc_tiling_on_sc=False),
        scratch_types=[
            pltpu.VMEM((_PER_W,), jnp.int32),
            pltpu.VMEM((_CHUNK, _D), jnp.float32),
            pltpu.VMEM((_CHUNK, _D), jnp.float32),
            pltpu.SemaphoreType.DMA,
            pltpu.SemaphoreType.DMA,
        ],
    )
    def k(idx_ref, tab_ref, out_ref, idx_v, buf0, buf1, gsem, osem):
        wid = lax.axis_index("s") * _NC + lax.axis_index("c")
        base = wid * _PER_W
        pltpu.sync_copy(idx_ref.at[pl.ds(base, _PER_W)], idx_v)

        def body(i, carry):
            copies = []
            for b, buf in ((0, buf0), (1, buf1)):
                c = i * 2 + b
                for g in range(_NG):
                    copies.append(
                        pltpu.async_copy(
                            tab_ref.at[
                                idx_v.at[pl.ds(c * _CHUNK + g * _GSZ, _GSZ)]
                            ],
                            buf.at[pl.ds(g * _GSZ, _GSZ)],
                            gsem,
                        )
                    )
            for cp in copies:
                cp.wait()
            stores = []
            for b, buf in ((0, buf0), (1, buf1)):
                c = i * 2 + b
                stores.append(
                    pltpu.async_copy(
                        buf,
                        out_ref.at[pl.ds(base + c * _CHUNK, _CHUNK)],
                        osem,
                    )
                )
            for s in stores:
                s.wait()
            return carry

        lax.fori_loop(0, _NCHUNK // 2, body, 0)

    return k(idx_hbm, tab_hbm)


def kernel(indices, tables):
    idx32 = indices.astype(jnp.int32)
    flat_idx = (idx32 + jnp.arange(_F, dtype=jnp.int32)[None, :] * _V).reshape(_N)
    big = tables.reshape(_F * _V, _D)
    out = _sc_gather(flat_idx, big)
    return out.reshape(_B, _F * _D)


# speedup vs baseline: 34.0017x; 34.0017x over previous
import functools

import jax
import jax.numpy as jnp
from jax import lax
from jax.experimental import pallas as pl
from jax.experimental.pallas import tpu as pltpu
from jax.experimental.pallas import tpu_sc as plsc

_B = 16384
_F = 26
_V = 100001
_D = 16

_NC = 2
_NS = 16
_FG = _F // 2
_CB = 8192
_NCH = _B // _CB


def _sc_lookup(idx_t, tab_t):
    mesh = plsc.VectorSubcoreMesh(core_axis_name="c", subcore_axis_name="s")

    @functools.partial(
        pl.kernel,
        mesh=mesh,
        out_type=jax.ShapeDtypeStruct((_F * _D, _B), jnp.float32),
        compiler_params=pltpu.CompilerParams(
            use_tc_tiling_on_sc=True, needs_layout_passes=False
        ),
        scratch_types=[
            pltpu.VMEM((_V,), jnp.float32),
            pltpu.VMEM((_CB,), jnp.int32),
            pltpu.VMEM((_CB,), jnp.float32),
        ],
    )
    def k(idx_ref, tab_ref, out_ref, row_v, idx_v, out_v):
        wid = lax.axis_index("s") * _NC + lax.axis_index("c")
        d = wid % _D
        f0 = (wid // _D) * _FG

        def feat_body(j, carry):
            f = f0 + j
            pltpu.sync_copy(tab_ref.at[f, d, :], row_v)

            def chunk_body(cb, carry2):
                b0 = cb * _CB
                pltpu.sync_copy(idx_ref.at[f, pl.ds(b0, _CB)], idx_v)

                def gather_body(i, carry3):
                    iv = idx_v[pl.ds(i * 16, 16)]
                    out_v[pl.ds(i * 16, 16)] = plsc.load_gather(row_v, [iv])
                    return carry3

                lax.fori_loop(0, _CB // 16, gather_body, 0)
                pltpu.sync_copy(out_v, out_ref.at[f * _D + d, pl.ds(b0, _CB)])
                return carry2

            return lax.fori_loop(0, _NCH, chunk_body, carry)

        lax.fori_loop(0, _FG, feat_body, 0)

    return k(idx_t, tab_t)


def kernel(indices, tables):
    idx_t = indices.astype(jnp.int32).T
    tab_t = jnp.transpose(tables, (0, 2, 1))
    out_t = _sc_lookup(idx_t, tab_t)
    return out_t.T.reshape(_B, _F * _D)


# speedup vs baseline: 38.7629x; 1.1400x over previous
import functools

import jax
import jax.numpy as jnp
from jax import lax
from jax.experimental import pallas as pl
from jax.experimental.pallas import tpu as pltpu
from jax.experimental.pallas import tpu_sc as plsc

_B = 16384
_F = 26
_V = 100001
_D = 16

_NC = 2
_NS = 16
_FG = _F // 2
_CB = 8192
_NCH = _B // _CB


def _sc_lookup(idx_t, tab_t):
    mesh = plsc.VectorSubcoreMesh(core_axis_name="c", subcore_axis_name="s")

    @functools.partial(
        pl.kernel,
        mesh=mesh,
        out_type=jax.ShapeDtypeStruct((_F * _D, _B), jnp.float32),
        compiler_params=pltpu.CompilerParams(
            use_tc_tiling_on_sc=True, needs_layout_passes=False
        ),
        scratch_types=[
            pltpu.VMEM((_V,), jnp.float32),
            pltpu.VMEM((_CB,), jnp.int32),
            pltpu.VMEM((_CB,), jnp.float32),
        ],
    )
    def k(idx_ref, tab_ref, out_ref, row_v, idx_v, out_v):
        wid = lax.axis_index("s") * _NC + lax.axis_index("c")
        d = wid % _D
        f0 = (wid // _D) * _FG

        def feat_body(j, carry):
            f = f0 + j
            pltpu.sync_copy(tab_ref.at[f, d, :], row_v)

            def chunk_body(cb, carry2):
                b0 = cb * _CB
                pltpu.sync_copy(idx_ref.at[f, pl.ds(b0, _CB)], idx_v)

                def gather_body(i, carry3):
                    for u in range(8):
                        o = i * 128 + u * 16
                        iv = idx_v[pl.ds(o, 16)]
                        out_v[pl.ds(o, 16)] = plsc.load_gather(row_v, [iv])
                    return carry3

                lax.fori_loop(0, _CB // 128, gather_body, 0)
                pltpu.sync_copy(out_v, out_ref.at[f * _D + d, pl.ds(b0, _CB)])
                return carry2

            return lax.fori_loop(0, _NCH, chunk_body, carry)

        lax.fori_loop(0, _FG, feat_body, 0)

    return k(idx_t, tab_t)


def kernel(indices, tables):
    idx_t = indices.astype(jnp.int32).T
    tab_t = jnp.transpose(tables, (0, 2, 1))
    out_t = _sc_lookup(idx_t, tab_t)
    return out_t.T.reshape(_B, _F * _D)


# speedup vs baseline: 44.3005x; 1.1429x over previous
import functools

import jax
import jax.numpy as jnp
from jax import lax
from jax.experimental import pallas as pl
from jax.experimental.pallas import tpu as pltpu
from jax.experimental.pallas import tpu_sc as plsc

_B = 16384
_F = 26
_V = 100001
_D = 16

_NC = 2
_NS = 16
_FG = _F // 2
_CB = 4096
_NCH = _B // _CB


def _sc_lookup(idx_t, tab_t):
    mesh = plsc.VectorSubcoreMesh(core_axis_name="c", subcore_axis_name="s")

    @functools.partial(
        pl.kernel,
        mesh=mesh,
        out_type=jax.ShapeDtypeStruct((_F * _D, _B), jnp.float32),
        compiler_params=pltpu.CompilerParams(
            use_tc_tiling_on_sc=True, needs_layout_passes=False
        ),
        scratch_types=[
            pltpu.VMEM((_V,), jnp.float32),
            pltpu.VMEM((_CB,), jnp.int32),
            pltpu.VMEM((_CB,), jnp.int32),
            pltpu.VMEM((_CB,), jnp.float32),
            pltpu.VMEM((_CB,), jnp.float32),
            pltpu.SemaphoreType.DMA,
            pltpu.SemaphoreType.DMA,
            pltpu.SemaphoreType.DMA,
        ],
    )
    def k(idx_ref, tab_ref, out_ref, row_v, idx0, idx1, out0, out1,
          isem, osem0, osem1):
        wid = lax.axis_index("s") * _NC + lax.axis_index("c")
        d = wid % _D
        f0 = (wid // _D) * _FG
        idxb = (idx0, idx1)
        outb = (out0, out1)
        osems = (osem0, osem1)

        def feat_body(j, carry):
            f = f0 + j
            pltpu.sync_copy(tab_ref.at[f, d, :], row_v)
            fetches = [
                pltpu.async_copy(idx_ref.at[f, pl.ds(0, _CB)], idx0, isem)
            ]
            stores = []
            for cb in range(_NCH):
                if cb + 1 < _NCH:
                    fetches.append(
                        pltpu.async_copy(
                            idx_ref.at[f, pl.ds((cb + 1) * _CB, _CB)],
                            idxb[(cb + 1) % 2],
                            isem,
                        )
                    )
                fetches.pop(0).wait()
                idx_v = idxb[cb % 2]
                out_v = outb[cb % 2]
                if cb >= 2:
                    stores[cb - 2].wait()

                def gather_body(i, carry3):
                    for u in range(8):
                        o = i * 128 + u * 16
                        iv = idx_v[pl.ds(o, 16)]
                        out_v[pl.ds(o, 16)] = plsc.load_gather(row_v, [iv])
                    return carry3

                lax.fori_loop(0, _CB // 128, gather_body, 0)
                stores.append(
                    pltpu.async_copy(
                        out_v,
                        out_ref.at[f * _D + d, pl.ds(cb * _CB, _CB)],
                        osems[cb % 2],
                    )
                )
            for s in stores[-2:]:
                s.wait()
            return carry

        lax.fori_loop(0, _FG, feat_body, 0)

    return k(idx_t, tab_t)


def kernel(indices, tables):
    idx_t = indices.astype(jnp.int32).T
    tab_t = jnp.transpose(tables, (0, 2, 1))
    out_t = _sc_lookup(idx_t, tab_t)
    return out_t.T.reshape(_B, _F * _D)
